# baseline (speedup 1.0000x reference)
import functools

import jax
import jax.numpy as jnp
from jax import lax
from jax.experimental import pallas as pl
from jax.experimental.pallas import tpu as pltpu
from jax.experimental.pallas import tpu_sc as plsc

N = 1024
E = 2048
F32 = jnp.float32
BF16 = jnp.bfloat16


def _dot(a, b, precision=None):
    return lax.dot_general(a, b, (((1,), (0,)), ((), ())),
                           precision=precision, preferred_element_type=F32)


def _gather_rows(table, idx):
    info = plsc.get_sparse_core_info()
    NC, NS = info.num_cores, info.num_subcores
    NW = NC * NS
    B = idx.shape[0]
    D = table.shape[1]
    bpw = B // NW
    mesh = plsc.VectorSubcoreMesh(core_axis_name="c", subcore_axis_name="s")

    @functools.partial(
        pl.kernel,
        out_type=jax.ShapeDtypeStruct((B, D), F32),
        mesh=mesh,
        scratch_types=[
            pltpu.VMEM((bpw,), jnp.int32),
            pltpu.VMEM((bpw, D), F32),
            pltpu.SemaphoreType.DMA,
        ],
    )
    def k(table_hbm, idx_hbm, out_hbm, idx_v, rows_v, sem):
        wid = lax.axis_index("s") * NC + lax.axis_index("c")
        base = wid * bpw
        pltpu.sync_copy(idx_hbm.at[pl.ds(base, bpw)], idx_v)
        pltpu.async_copy(table_hbm.at[idx_v], rows_v, sem).wait()
        pltpu.sync_copy(rows_v, out_hbm.at[pl.ds(base, bpw)])

    return k(table, idx)


def _h_kernel(ea_ref, w1_ref, b1_ref, h_ref):
    h_ref[...] = jax.nn.leaky_relu(
        _dot(ea_ref[...].astype(BF16), w1_ref[...].astype(BF16))
        + b1_ref[...], 0.01).astype(BF16)


def _edge_hidden_all(edge_attr, W1s, b1s):
    W1 = jnp.concatenate(W1s, axis=1)
    b1 = jnp.concatenate(b1s)
    K = W1.shape[1]
    return pl.pallas_call(
        _h_kernel,
        out_shape=jax.ShapeDtypeStruct((E, K), BF16),
    )(edge_attr, W1, b1.reshape(1, K))


def _mm_kernel(hb_ref, xjt_ref, w2_ref, b2_ref, out_ref, *, ci, eb, out_ch):
    c = pl.program_id(0)
    e = pl.program_id(1)
    esl = pl.ds(e * eb, eb)

    @pl.when(c == 0)
    def _():
        out_ref[esl, :] = jnp.zeros_like(out_ref[esl, :])

    hblk = hb_ref[esl, :]
    wdt = _dot(hblk, w2_ref[...]) + b2_ref[...]
    wdf = wdt.astype(BF16).astype(F32)
    xjs = xjt_ref[pl.ds(c * ci, ci), esl]
    xjf = xjs.astype(BF16).astype(F32).T
    acc = out_ref[esl, :]
    for j in range(ci):
        acc = acc + xjf[:, j:j + 1] * wdf[:, j * out_ch:(j + 1) * out_ch]
    out_ref[esl, :] = acc


def _edge_messages(hall, koff, K, xjt, W2, b2, in_ch, out_ch, ci, eb):
    C = ci * out_ch
    nc = in_ch // ci
    ne = E // eb
    xw = xjt.shape[0]
    kb = koff // K

    return pl.pallas_call(
        functools.partial(_mm_kernel, ci=ci, eb=eb, out_ch=out_ch),
        grid=(nc, ne),
        in_specs=[
            pl.BlockSpec((E, K), lambda c, e: (0, kb)),
            pl.BlockSpec((xw, E), lambda c, e: (0, 0)),
            pl.BlockSpec((K, C), lambda c, e: (0, c)),
            pl.BlockSpec((1, C), lambda c, e: (0, c)),
        ],
        out_specs=pl.BlockSpec((E, out_ch), lambda c, e: (0, 0)),
        out_shape=jax.ShapeDtypeStruct((E, out_ch), F32),
        compiler_params=pltpu.CompilerParams(
            dimension_semantics=("arbitrary", "arbitrary")),
    )(hall, xjt, W2.astype(BF16), b2.reshape(1, in_ch * out_ch))


def _agg_kernel(dst_ref, msgt_ref, x_ref, root_ref, bias_ref, o_ref, *, nb,
                gumbel_ref=None):
    i = pl.program_id(0)
    dstv = dst_ref[...]
    iot = lax.broadcasted_iota(jnp.int32, (nb, E), 0) + i * nb
    P = (iot == dstv).astype(F32)
    s = _dot(P, msgt_ref[...], precision=lax.Precision.HIGHEST)
    c = jnp.maximum(jnp.sum(P, axis=1, keepdims=True), 1.0)
    xr = _dot(x_ref[...].astype(BF16), root_ref[...].astype(BF16))
    d = jax.nn.leaky_relu(s / c + xr + bias_ref[...], 0.01)
    if gumbel_ref is None:
        o_ref[...] = d
        return
    v = d + gumbel_ref[...]
    cols = v.shape[1]
    m = jnp.max(v, axis=1, keepdims=True)
    oi = lax.broadcasted_iota(jnp.int32, v.shape, 1)
    first = jnp.min(jnp.where(v == m, oi, cols), axis=1, keepdims=True)
    o_ref[...] = (oi == first).astype(F32)


def _aggregate(dst2d, msgt, x, root, bias, g=None):
    in_ch, out_ch = root.shape
    nb = 256
    if g is None:
        body = functools.partial(_agg_kernel, nb=nb)
    else:
        def body(dst_ref, msgt_ref, x_ref, root_ref, bias_ref, g_ref, o_ref):
            return _agg_kernel(dst_ref, msgt_ref, x_ref, root_ref, bias_ref,
                               o_ref, nb=nb, gumbel_ref=g_ref)
    in_specs = [
        pl.BlockSpec((1, E), lambda i: (0, 0)),
        pl.BlockSpec((E, out_ch), lambda i: (0, 0)),
        pl.BlockSpec((nb, in_ch), lambda i: (i, 0)),
        pl.BlockSpec((in_ch, out_ch), lambda i: (0, 0)),
        pl.BlockSpec((1, out_ch), lambda i: (0, 0)),
    ]
    args = [dst2d, msgt, x, root, bias.reshape(1, out_ch)]
    if g is not None:
        in_specs.append(pl.BlockSpec((nb, out_ch), lambda i: (i, 0)))
        args.append(g)
    return pl.pallas_call(
        body,
        grid=(N // nb,),
        in_specs=in_specs,
        out_specs=pl.BlockSpec((nb, out_ch), lambda i: (i, 0)),
        out_shape=jax.ShapeDtypeStruct((N, out_ch), F32),
    )(*args)


def _layer(x_cur, src2, dst2d, hall, koff, K, W2, b2, root, bias,
           in_ch, out_ch, ci, eb, g=None, gather_src=None):
    xj = _gather_rows(gather_src if gather_src is not None else x_cur, src2)
    xjt = xj.T
    msg = _edge_messages(hall, koff, K, xjt, W2, b2, in_ch, out_ch,
                         ci=ci, eb=eb)
    return _aggregate(dst2d, msg, x_cur, root, bias, g)


def kernel(x, edge_index, edge_attr, epoch,
           nn1_W1, nn1_b1, nn1_W2, nn1_b2, root1, bias1,
           nn2_W1, nn2_b1, nn2_W2, nn2_b2, root2, bias2,
           nn3_W1, nn3_b1, nn3_W2, nn3_b2, root3, bias3):
    src = edge_index[0]
    dst2d = edge_index[1].reshape(1, E)
    g = jax.random.gumbel(jax.random.key(42), (N, 64), dtype=F32)
    hall = _edge_hidden_all(edge_attr, (nn1_W1, nn2_W1, nn3_W1),
                            (nn1_b1, nn2_b1, nn3_b1))

    xp = jnp.pad(x, ((0, 0), (0, 64)))
    d1 = _layer(x, src, dst2d, hall, 0, 512, nn1_W2, nn1_b2,
                root1, bias1, 64, 512, ci=8, eb=1024, gather_src=xp)
    d2 = _layer(d1, src, dst2d, hall, 512, 256, nn2_W2, nn2_b2,
                root2, bias2, 512, 256, ci=16, eb=1024)
    return _layer(d2, src, dst2d, hall, 768, 128, nn3_W2, nn3_b2,
                  root3, bias3, 256, 64, ci=32, eb=1024, g=g)


# speedup vs baseline: 1.1828x; 1.1828x over previous
import functools

import jax
import jax.numpy as jnp
from jax import lax
from jax.experimental import pallas as pl
from jax.experimental.pallas import tpu as pltpu
from jax.experimental.pallas import tpu_sc as plsc

N = 1024
E = 2048
F32 = jnp.float32
BF16 = jnp.bfloat16


def _dot(a, b, precision=None):
    return lax.dot_general(a, b, (((1,), (0,)), ((), ())),
                           precision=precision, preferred_element_type=F32)


def _gather_rows(table, idx):
    info = plsc.get_sparse_core_info()
    NC, NS = info.num_cores, info.num_subcores
    NW = NC * NS
    B = idx.shape[0]
    D = table.shape[1]
    bpw = B // NW
    mesh = plsc.VectorSubcoreMesh(core_axis_name="c", subcore_axis_name="s")

    @functools.partial(
        pl.kernel,
        out_type=jax.ShapeDtypeStruct((B, D), F32),
        mesh=mesh,
        scratch_types=[
            pltpu.VMEM((bpw,), jnp.int32),
            pltpu.VMEM((bpw, D), F32),
            pltpu.SemaphoreType.DMA,
        ],
    )
    def k(table_hbm, idx_hbm, out_hbm, idx_v, rows_v, sem):
        wid = lax.axis_index("s") * NC + lax.axis_index("c")
        base = wid * bpw
        pltpu.sync_copy(idx_hbm.at[pl.ds(base, bpw)], idx_v)
        pltpu.async_copy(table_hbm.at[idx_v], rows_v, sem).wait()
        pltpu.sync_copy(rows_v, out_hbm.at[pl.ds(base, bpw)])

    return k(table, idx)


def _h_kernel(ea_ref, w1_ref, b1_ref, h_ref):
    h_ref[...] = jax.nn.leaky_relu(
        _dot(ea_ref[...].astype(BF16), w1_ref[...].astype(BF16))
        + b1_ref[...], 0.01).astype(BF16)


def _edge_hidden_all(edge_attr, W1s, b1s):
    W1 = jnp.concatenate(W1s, axis=1)
    b1 = jnp.concatenate(b1s)
    K = W1.shape[1]
    return pl.pallas_call(
        _h_kernel,
        out_shape=jax.ShapeDtypeStruct((E, K), BF16),
    )(edge_attr, W1, b1.reshape(1, K))


def _mm_kernel(hb_ref, xjt_ref, w2_ref, b2_ref, dst_ref, x_ref, root_ref,
               bias_ref, g_ref, d_ref, msg_ref, *, ci, eb, out_ch, nc, ne,
               final):
    c = pl.program_id(0)
    e = pl.program_id(1)
    esl = pl.ds(e * eb, eb)

    @pl.when(c == 0)
    def _():
        msg_ref[esl, :] = jnp.zeros_like(msg_ref[esl, :])

    hblk = hb_ref[esl, :]
    w2b = w2_ref[...].astype(BF16)
    wdt = _dot(hblk, w2b) + b2_ref[...]
    wdf = wdt.astype(BF16).astype(F32)
    xjs = xjt_ref[pl.ds(c * ci, ci), esl]
    xjf = xjs.astype(BF16).astype(F32).T
    acc = msg_ref[esl, :]
    for j in range(ci):
        acc = acc + xjf[:, j:j + 1] * wdf[:, j * out_ch:(j + 1) * out_ch]
    msg_ref[esl, :] = acc

    @pl.when((c == nc - 1) & (e == ne - 1))
    def _():
        dstv = dst_ref[...]
        msg = msg_ref[...]
        nb = 256
        for i in range(N // nb):
            iot = lax.broadcasted_iota(jnp.int32, (nb, E), 0) + i * nb
            P = (iot == dstv).astype(F32)
            sagg = _dot(P, msg, precision=lax.Precision.HIGHEST)
            cnt = jnp.maximum(jnp.sum(P, axis=1, keepdims=True), 1.0)
            nsl = pl.ds(i * nb, nb)
            xr = _dot(x_ref[nsl, :].astype(BF16), root_ref[...].astype(BF16))
            d = jax.nn.leaky_relu(sagg / cnt + xr + bias_ref[...], 0.01)
            if not final:
                d_ref[nsl, :] = d
            else:
                v = d + g_ref[nsl, :]
                m = jnp.max(v, axis=1, keepdims=True)
                oi = lax.broadcasted_iota(jnp.int32, v.shape, 1)
                first = jnp.min(jnp.where(v == m, oi, v.shape[1]),
                                axis=1, keepdims=True)
                d_ref[nsl, :] = (oi == first).astype(F32)


def _nnconv_layer(hall, koff, K, xjt, W2, b2, dst2d, x, root, bias, g,
                  in_ch, out_ch, ci, eb):
    C = ci * out_ch
    nc = in_ch // ci
    ne = E // eb
    xw = xjt.shape[0]
    kb = koff // K
    final = g is not None
    if g is None:
        g = jnp.zeros((N, out_ch), F32)

    return pl.pallas_call(
        functools.partial(_mm_kernel, ci=ci, eb=eb, out_ch=out_ch,
                          nc=nc, ne=ne, final=final),
        grid=(nc, ne),
        in_specs=[
            pl.BlockSpec((E, K), lambda c, e: (0, kb)),
            pl.BlockSpec((xw, E), lambda c, e: (0, 0)),
            pl.BlockSpec((K, C), lambda c, e: (0, c)),
            pl.BlockSpec((1, C), lambda c, e: (0, c)),
            pl.BlockSpec((1, E), lambda c, e: (0, 0)),
            pl.BlockSpec((N, root.shape[0]), lambda c, e: (0, 0)),
            pl.BlockSpec(root.shape, lambda c, e: (0, 0)),
            pl.BlockSpec((1, out_ch), lambda c, e: (0, 0)),
            pl.BlockSpec((N, out_ch), lambda c, e: (0, 0)),
        ],
        out_specs=pl.BlockSpec((N, out_ch), lambda c, e: (0, 0)),
        out_shape=jax.ShapeDtypeStruct((N, out_ch), F32),
        scratch_shapes=[pltpu.VMEM((E, out_ch), F32)],
        compiler_params=pltpu.CompilerParams(
            dimension_semantics=("arbitrary", "arbitrary")),
    )(hall, xjt, W2, b2.reshape(1, in_ch * out_ch), dst2d, x, root,
      bias.reshape(1, out_ch), g)


def _layer(x_cur, src2, dst2d, hall, koff, K, W2, b2, root, bias,
           in_ch, out_ch, ci, eb, g=None, gather_src=None):
    xj = _gather_rows(gather_src if gather_src is not None else x_cur, src2)
    xjt = xj.T
    return _nnconv_layer(hall, koff, K, xjt, W2, b2, dst2d, x_cur, root,
                         bias, g, in_ch, out_ch, ci=ci, eb=eb)


def kernel(x, edge_index, edge_attr, epoch,
           nn1_W1, nn1_b1, nn1_W2, nn1_b2, root1, bias1,
           nn2_W1, nn2_b1, nn2_W2, nn2_b2, root2, bias2,
           nn3_W1, nn3_b1, nn3_W2, nn3_b2, root3, bias3):
    src = edge_index[0]
    dst2d = edge_index[1].reshape(1, E)
    g = jax.random.gumbel(jax.random.key(42), (N, 64), dtype=F32)
    hall = _edge_hidden_all(edge_attr, (nn1_W1, nn2_W1, nn3_W1),
                            (nn1_b1, nn2_b1, nn3_b1))

    xp = jnp.pad(x, ((0, 0), (0, 64)))
    d1 = _layer(x, src, dst2d, hall, 0, 512, nn1_W2, nn1_b2,
                root1, bias1, 64, 512, ci=8, eb=1024, gather_src=xp)
    d2 = _layer(d1, src, dst2d, hall, 512, 256, nn2_W2, nn2_b2,
                root2, bias2, 512, 256, ci=16, eb=1024)
    return _layer(d2, src, dst2d, hall, 768, 128, nn3_W2, nn3_b2,
                  root3, bias3, 256, 64, ci=32, eb=1024, g=g)


# speedup vs baseline: 1.2084x; 1.0217x over previous
import functools

import jax
import jax.numpy as jnp
from jax import lax
from jax.experimental import pallas as pl
from jax.experimental.pallas import tpu as pltpu
from jax.experimental.pallas import tpu_sc as plsc

N = 1024
E = 2048
F32 = jnp.float32
BF16 = jnp.bfloat16


def _dot(a, b, precision=None):
    return lax.dot_general(a, b, (((1,), (0,)), ((), ())),
                           precision=precision, preferred_element_type=F32)


def _gather_rows(table, idx):
    info = plsc.get_sparse_core_info()
    NC, NS = info.num_cores, info.num_subcores
    NW = NC * NS
    B = idx.shape[0]
    D = table.shape[1]
    bpw = B // NW
    mesh = plsc.VectorSubcoreMesh(core_axis_name="c", subcore_axis_name="s")

    @functools.partial(
        pl.kernel,
        out_type=jax.ShapeDtypeStruct((B, D), F32),
        mesh=mesh,
        scratch_types=[
            pltpu.VMEM((bpw,), jnp.int32),
            pltpu.VMEM((bpw, D), F32),
            pltpu.SemaphoreType.DMA,
        ],
    )
    def k(table_hbm, idx_hbm, out_hbm, idx_v, rows_v, sem):
        wid = lax.axis_index("s") * NC + lax.axis_index("c")
        base = wid * bpw
        pltpu.sync_copy(idx_hbm.at[pl.ds(base, bpw)], idx_v)
        pltpu.async_copy(table_hbm.at[idx_v], rows_v, sem).wait()
        pltpu.sync_copy(rows_v, out_hbm.at[pl.ds(base, bpw)])

    return k(table, idx)


def _h_kernel(ea_ref, w1_ref, b1_ref, h_ref):
    h_ref[...] = jax.nn.leaky_relu(
        _dot(ea_ref[...].astype(BF16), w1_ref[...].astype(BF16))
        + b1_ref[...], 0.01).astype(BF16)


def _edge_hidden_all(edge_attr, W1s, b1s):
    W1 = jnp.concatenate(W1s, axis=1)
    b1 = jnp.concatenate(b1s)
    K = W1.shape[1]
    return pl.pallas_call(
        _h_kernel,
        out_shape=jax.ShapeDtypeStruct((E, K), BF16),
    )(edge_attr, W1, b1.reshape(1, K))


def _mm_kernel(hb_ref, xjt_ref, w2_ref, b2_ref, dst_ref, x_ref, root_ref,
               bias_ref, g_ref, d_ref, msg_ref, *, ci, eb, out_ch, nc, ne,
               final):
    c = pl.program_id(0)
    e = pl.program_id(1)
    esl = pl.ds(e * eb, eb)

    @pl.when(c == 0)
    def _():
        msg_ref[esl, :] = jnp.zeros_like(msg_ref[esl, :])

    hblk = hb_ref[esl, :]
    xjs = xjt_ref[pl.ds(c * ci, ci), esl]
    xjf = xjs.astype(BF16).astype(F32).T
    acc = msg_ref[esl, :]
    for j in range(ci):
        w2b = w2_ref[:, j * out_ch:(j + 1) * out_ch].astype(BF16)
        wdt = _dot(hblk, w2b) + b2_ref[:, j * out_ch:(j + 1) * out_ch]
        wdf = wdt.astype(BF16).astype(F32)
        acc = acc + xjf[:, j:j + 1] * wdf
    msg_ref[esl, :] = acc

    @pl.when((c == nc - 1) & (e == ne - 1))
    def _():
        dstv = dst_ref[...]
        msg = msg_ref[...]
        nb = 256
        for i in range(N // nb):
            iot = lax.broadcasted_iota(jnp.int32, (nb, E), 0) + i * nb
            P = (iot == dstv).astype(F32)
            sagg = _dot(P, msg, precision=lax.Precision.HIGHEST)
            cnt = jnp.maximum(jnp.sum(P, axis=1, keepdims=True), 1.0)
            nsl = pl.ds(i * nb, nb)
            xr = _dot(x_ref[nsl, :].astype(BF16), root_ref[...].astype(BF16))
            d = jax.nn.leaky_relu(sagg / cnt + xr + bias_ref[...], 0.01)
            if not final:
                d_ref[nsl, :] = d
            else:
                v = d + g_ref[nsl, :]
                m = jnp.max(v, axis=1, keepdims=True)
                oi = lax.broadcasted_iota(jnp.int32, v.shape, 1)
                first = jnp.min(jnp.where(v == m, oi, v.shape[1]),
                                axis=1, keepdims=True)
                d_ref[nsl, :] = (oi == first).astype(F32)


def _nnconv_layer(hall, koff, K, xjt, W2, b2, dst2d, x, root, bias, g,
                  in_ch, out_ch, ci, eb):
    C = ci * out_ch
    nc = in_ch // ci
    ne = E // eb
    xw = xjt.shape[0]
    kb = koff // K
    final = g is not None
    if g is None:
        g = jnp.zeros((N, out_ch), F32)

    return pl.pallas_call(
        functools.partial(_mm_kernel, ci=ci, eb=eb, out_ch=out_ch,
                          nc=nc, ne=ne, final=final),
        grid=(nc, ne),
        in_specs=[
            pl.BlockSpec((E, K), lambda c, e: (0, kb)),
            pl.BlockSpec((xw, E), lambda c, e: (0, 0)),
            pl.BlockSpec((K, C), lambda c, e: (0, c)),
            pl.BlockSpec((1, C), lambda c, e: (0, c)),
            pl.BlockSpec((1, E), lambda c, e: (0, 0)),
            pl.BlockSpec((N, root.shape[0]), lambda c, e: (0, 0)),
            pl.BlockSpec(root.shape, lambda c, e: (0, 0)),
            pl.BlockSpec((1, out_ch), lambda c, e: (0, 0)),
            pl.BlockSpec((N, out_ch), lambda c, e: (0, 0)),
        ],
        out_specs=pl.BlockSpec((N, out_ch), lambda c, e: (0, 0)),
        out_shape=jax.ShapeDtypeStruct((N, out_ch), F32),
        scratch_shapes=[pltpu.VMEM((E, out_ch), F32)],
        compiler_params=pltpu.CompilerParams(
            dimension_semantics=("arbitrary", "arbitrary")),
    )(hall, xjt, W2, b2.reshape(1, in_ch * out_ch), dst2d, x, root,
      bias.reshape(1, out_ch), g)


def _layer(x_cur, src2, dst2d, hall, koff, K, W2, b2, root, bias,
           in_ch, out_ch, ci, eb, g=None, gather_src=None):
    xj = _gather_rows(gather_src if gather_src is not None else x_cur, src2)
    xjt = xj.T
    return _nnconv_layer(hall, koff, K, xjt, W2, b2, dst2d, x_cur, root,
                         bias, g, in_ch, out_ch, ci=ci, eb=eb)


def kernel(x, edge_index, edge_attr, epoch,
           nn1_W1, nn1_b1, nn1_W2, nn1_b2, root1, bias1,
           nn2_W1, nn2_b1, nn2_W2, nn2_b2, root2, bias2,
           nn3_W1, nn3_b1, nn3_W2, nn3_b2, root3, bias3):
    src = edge_index[0]
    dst2d = edge_index[1].reshape(1, E)
    g = jax.random.gumbel(jax.random.key(42), (N, 64), dtype=F32)
    hall = _edge_hidden_all(edge_attr, (nn1_W1, nn2_W1, nn3_W1),
                            (nn1_b1, nn2_b1, nn3_b1))

    xp = jnp.pad(x, ((0, 0), (0, 64)))
    d1 = _layer(x, src, dst2d, hall, 0, 512, nn1_W2, nn1_b2,
                root1, bias1, 64, 512, ci=8, eb=1024, gather_src=xp)
    d2 = _layer(d1, src, dst2d, hall, 512, 256, nn2_W2, nn2_b2,
                root2, bias2, 512, 256, ci=16, eb=1024)
    return _layer(d2, src, dst2d, hall, 768, 128, nn3_W2, nn3_b2,
                  root3, bias3, 256, 64, ci=32, eb=1024, g=g)


# speedup vs baseline: 1.2378x; 1.0243x over previous
import functools

import jax
import jax.numpy as jnp
from jax import lax
from jax.experimental import pallas as pl
from jax.experimental.pallas import tpu as pltpu
from jax.experimental.pallas import tpu_sc as plsc

N = 1024
E = 2048
F32 = jnp.float32
BF16 = jnp.bfloat16


def _dot(a, b, precision=None):
    return lax.dot_general(a, b, (((1,), (0,)), ((), ())),
                           precision=precision, preferred_element_type=F32)


def _gather_rows(table, idx):
    info = plsc.get_sparse_core_info()
    NC, NS = info.num_cores, info.num_subcores
    NW = NC * NS
    B = idx.shape[0]
    D = table.shape[1]
    bpw = B // NW
    mesh = plsc.VectorSubcoreMesh(core_axis_name="c", subcore_axis_name="s")

    @functools.partial(
        pl.kernel,
        out_type=jax.ShapeDtypeStruct((B, D), F32),
        mesh=mesh,
        scratch_types=[
            pltpu.VMEM((bpw,), jnp.int32),
            pltpu.VMEM((bpw, D), F32),
            pltpu.SemaphoreType.DMA,
        ],
    )
    def k(table_hbm, idx_hbm, out_hbm, idx_v, rows_v, sem):
        wid = lax.axis_index("s") * NC + lax.axis_index("c")
        base = wid * bpw
        pltpu.sync_copy(idx_hbm.at[pl.ds(base, bpw)], idx_v)
        pltpu.async_copy(table_hbm.at[idx_v], rows_v, sem).wait()
        pltpu.sync_copy(rows_v, out_hbm.at[pl.ds(base, bpw)])

    return k(table, idx)


def _h_kernel(ea_ref, w1_ref, b1_ref, h_ref):
    h_ref[...] = jax.nn.leaky_relu(
        _dot(ea_ref[...].astype(BF16), w1_ref[...].astype(BF16))
        + b1_ref[...], 0.01).astype(BF16)


def _edge_hidden_all(edge_attr, W1s, b1s):
    W1 = jnp.concatenate(W1s, axis=1)
    b1 = jnp.concatenate(b1s)
    K = W1.shape[1]
    return pl.pallas_call(
        _h_kernel,
        out_shape=jax.ShapeDtypeStruct((E, K), BF16),
    )(edge_attr, W1, b1.reshape(1, K))


def _mm_kernel(hb_ref, xjt_ref, w2_ref, b2_ref, dst_ref, x_ref, root_ref,
               bias_ref, g_ref, d_ref, msg_ref, *, ci, eb, out_ch, nc, ne,
               final):
    c = pl.program_id(0)
    e = pl.program_id(1)
    esl = pl.ds(e * eb, eb)

    @pl.when(c == 0)
    def _():
        msg_ref[esl, :] = jnp.zeros_like(msg_ref[esl, :])

    hblk = hb_ref[esl, :]
    xjs = xjt_ref[pl.ds(c * ci, ci), esl]
    xjf = xjs.astype(BF16).astype(F32).T
    acc = msg_ref[esl, :]
    for j in range(ci):
        w2b = w2_ref[:, j * out_ch:(j + 1) * out_ch].astype(BF16)
        wdt = _dot(hblk, w2b) + b2_ref[:, j * out_ch:(j + 1) * out_ch]
        wdf = wdt.astype(BF16).astype(F32)
        acc = acc + xjf[:, j:j + 1] * wdf
    msg_ref[esl, :] = acc

    @pl.when((c == nc - 1) & (e == ne - 1))
    def _():
        dstv = dst_ref[...]
        msg = msg_ref[...]
        nb = 256
        for i in range(N // nb):
            iot = lax.broadcasted_iota(jnp.int32, (nb, E), 0) + i * nb
            P = (iot == dstv).astype(F32)
            sagg = _dot(P, msg, precision=lax.Precision.HIGHEST)
            cnt = jnp.maximum(jnp.sum(P, axis=1, keepdims=True), 1.0)
            nsl = pl.ds(i * nb, nb)
            xr = _dot(x_ref[nsl, :].astype(BF16), root_ref[...].astype(BF16))
            d = jax.nn.leaky_relu(sagg / cnt + xr + bias_ref[...], 0.01)
            if not final:
                d_ref[nsl, :] = d
            else:
                v = d + g_ref[nsl, :]
                m = jnp.max(v, axis=1, keepdims=True)
                oi = lax.broadcasted_iota(jnp.int32, v.shape, 1)
                first = jnp.min(jnp.where(v == m, oi, v.shape[1]),
                                axis=1, keepdims=True)
                d_ref[nsl, :] = (oi == first).astype(F32)


def _nnconv_layer(hall, koff, K, xjt, W2, b2, dst2d, x, root, bias, g,
                  in_ch, out_ch, ci, eb):
    C = ci * out_ch
    nc = in_ch // ci
    ne = E // eb
    xw = xjt.shape[0]
    kb = koff // K
    final = g is not None
    if g is None:
        g = jnp.zeros((N, out_ch), F32)

    return pl.pallas_call(
        functools.partial(_mm_kernel, ci=ci, eb=eb, out_ch=out_ch,
                          nc=nc, ne=ne, final=final),
        grid=(nc, ne),
        in_specs=[
            pl.BlockSpec((E, K), lambda c, e: (0, kb)),
            pl.BlockSpec((xw, E), lambda c, e: (0, 0)),
            pl.BlockSpec((K, C), lambda c, e: (0, c)),
            pl.BlockSpec((1, C), lambda c, e: (0, c)),
            pl.BlockSpec((1, E), lambda c, e: (0, 0)),
            pl.BlockSpec((N, root.shape[0]), lambda c, e: (0, 0)),
            pl.BlockSpec(root.shape, lambda c, e: (0, 0)),
            pl.BlockSpec((1, out_ch), lambda c, e: (0, 0)),
            pl.BlockSpec((N, out_ch), lambda c, e: (0, 0)),
        ],
        out_specs=pl.BlockSpec((N, out_ch), lambda c, e: (0, 0)),
        out_shape=jax.ShapeDtypeStruct((N, out_ch), F32),
        scratch_shapes=[pltpu.VMEM((E, out_ch), F32)],
        compiler_params=pltpu.CompilerParams(
            dimension_semantics=("arbitrary", "arbitrary")),
    )(hall, xjt, W2, b2.reshape(1, in_ch * out_ch), dst2d, x, root,
      bias.reshape(1, out_ch), g)


def _layer(x_cur, src2, dst2d, hall, koff, K, W2, b2, root, bias,
           in_ch, out_ch, ci, eb, g=None, gather_src=None):
    xj = _gather_rows(gather_src if gather_src is not None else x_cur, src2)
    xjt = xj.T
    return _nnconv_layer(hall, koff, K, xjt, W2, b2, dst2d, x_cur, root,
                         bias, g, in_ch, out_ch, ci=ci, eb=eb)


def kernel(x, edge_index, edge_attr, epoch,
           nn1_W1, nn1_b1, nn1_W2, nn1_b2, root1, bias1,
           nn2_W1, nn2_b1, nn2_W2, nn2_b2, root2, bias2,
           nn3_W1, nn3_b1, nn3_W2, nn3_b2, root3, bias3):
    src = edge_index[0]
    dst2d = edge_index[1].reshape(1, E)
    g = jax.random.gumbel(jax.random.key(42), (N, 64), dtype=F32)
    hall = _edge_hidden_all(edge_attr, (nn1_W1, nn2_W1, nn3_W1),
                            (nn1_b1, nn2_b1, nn3_b1))

    xp = jnp.pad(x, ((0, 0), (0, 64)))
    d1 = _layer(x, src, dst2d, hall, 0, 512, nn1_W2, nn1_b2,
                root1, bias1, 64, 512, ci=8, eb=2048, gather_src=xp)
    d2 = _layer(d1, src, dst2d, hall, 512, 256, nn2_W2, nn2_b2,
                root2, bias2, 512, 256, ci=16, eb=2048)
    return _layer(d2, src, dst2d, hall, 768, 128, nn3_W2, nn3_b2,
                  root3, bias3, 256, 64, ci=32, eb=2048, g=g)
